# Initial kernel scaffold; baseline (speedup 1.0000x reference)
import functools

import jax
import jax.numpy as jnp
from jax import lax
from jax.experimental import pallas as pl
from jax.experimental.pallas import tpu as pltpu
from jax.experimental.pallas import tpu_sc as plsc

NC = 2
NS = 16
NW = NC * NS
CH = 128


def _round_up(a, b):
    return (a + b - 1) // b * b


def _sc_mesh():
    return plsc.VectorSubcoreMesh(core_axis_name="c", subcore_axis_name="s")


def _make_deg_kernel(n_acc, ept):
    @functools.partial(
        pl.kernel,
        out_type=jax.ShapeDtypeStruct((NW * n_acc,), jnp.float32),
        mesh=_sc_mesh(),
        scratch_types=[
            pltpu.VMEM((n_acc,), jnp.float32),
            pltpu.VMEM((ept,), jnp.int32),
        ],
    )
    def deg_kernel(dst_hbm, out_hbm, hist_v, dste_v):
        cid = lax.axis_index("c")
        sid = lax.axis_index("s")
        wid = cid * NS + sid
        zero = jnp.zeros((16,), jnp.float32)

        def zbody(i, carry):
            hist_v[pl.ds(i * 16, 16)] = zero
            return carry

        lax.fori_loop(0, n_acc // 16, zbody, 0)

        pltpu.sync_copy(dst_hbm.at[pl.ds(wid * ept, ept)], dste_v)
        ones = jnp.ones((16,), jnp.float32)

        def ebody(i, carry):
            idx = dste_v[pl.ds(i * 16, 16)]
            plsc.addupdate_scatter(hist_v, [idx], ones)
            return carry

        lax.fori_loop(0, ept // 16, ebody, 0)

        pltpu.sync_copy(hist_v, out_hbm.at[pl.ds(wid * n_acc, n_acc)])

    return deg_kernel


def _make_agg_kernel(n_acc, ept, nch, f):
    rpt = n_acc // NS

    @functools.partial(
        pl.kernel,
        out_type=jax.ShapeDtypeStruct((NC * n_acc, f), jnp.float32),
        mesh=_sc_mesh(),
        scratch_types=[
            pltpu.VMEM_SHARED((n_acc, f), jnp.float32),
            pltpu.VMEM((rpt, f), jnp.float32),
            pltpu.VMEM((CH,), jnp.int32),
            pltpu.VMEM((1, CH), jnp.int32),
            pltpu.VMEM((CH, f), jnp.float32),
            pltpu.SemaphoreType.DMA,
        ],
    )
    def agg_kernel(g_hbm, src_hbm, dst_hbm, out_hbm,
                   acc_sh, zrow_v, sidx_v, didx_v, rows_v, sem):
        cid = lax.axis_index("c")
        sid = lax.axis_index("s")
        wid = cid * NS + sid
        zero = jnp.zeros((16,), jnp.float32)

        def zbody(r, carry):
            for j in range(f // 16):
                zrow_v[r, pl.ds(j * 16, 16)] = zero
            return carry

        lax.fori_loop(0, rpt, zbody, 0)
        pltpu.sync_copy(zrow_v, acc_sh.at[pl.ds(sid * rpt, rpt)])
        plsc.subcore_barrier()

        ebase = wid * ept

        def chbody(ci, carry):
            b = ebase + ci * CH
            pltpu.sync_copy(src_hbm.at[pl.ds(b, CH)], sidx_v)
            pltpu.sync_copy(dst_hbm.at[pl.ds(b, CH)], didx_v.at[0])
            pltpu.async_copy(g_hbm.at[sidx_v], rows_v, sem).wait()
            pltpu.sync_copy(rows_v, acc_sh.at[didx_v.at[0]], add=True)
            return carry

        lax.fori_loop(0, nch, chbody, 0)
        plsc.subcore_barrier()

        pltpu.sync_copy(
            acc_sh.at[pl.ds(sid * rpt, rpt)],
            out_hbm.at[pl.ds(cid * n_acc + sid * rpt, rpt)],
        )

    return agg_kernel


def _deg_dinv(hist_blk):
    deg = jnp.sum(hist_blk, axis=0) + 1.0
    return lax.rsqrt(deg)


def _tc1_body(x_ref, w_ref, hist_ref, g1_ref):
    dinv = _deg_dinv(hist_ref[...])
    h = jnp.dot(x_ref[...], w_ref[...], preferred_element_type=jnp.float32)
    g1_ref[...] = h * dinv[:, None]


def _tc2_body(p_ref, g1_ref, hist_ref, w_ref, b_ref, g2_ref):
    dinv = _deg_dinv(hist_ref[...])
    s = (p_ref[0] + p_ref[1] + g1_ref[...]) * dinv[:, None] + b_ref[...]
    a = jnp.maximum(s, 0.0)
    h2 = jnp.dot(a, w_ref[...], preferred_element_type=jnp.float32)
    g2_ref[...] = h2 * dinv[:, None]


def _tc3_body(q_ref, g2_ref, hist_ref, b_ref, out_ref):
    dinv = _deg_dinv(hist_ref[...])
    z = (q_ref[0] + q_ref[1] + g2_ref[...]) * dinv[:, None] + b_ref[...]
    m = jnp.max(z, axis=1, keepdims=True)
    lse = jnp.log(jnp.sum(jnp.exp(z - m), axis=1, keepdims=True))
    out_ref[...] = z - m - lse


def kernel(x, edge_idx, W1, b1, W2, b2):
    n, d = x.shape
    h = W1.shape[1]
    c = W2.shape[1]
    e = edge_idx.shape[1]

    n_acc = _round_up(n + 1, 1024)
    br = 1024
    e_pad = _round_up(e, NW * CH)
    ept = e_pad // NW
    nch = ept // CH

    src = edge_idx[0]
    dst = edge_idx[1]
    pad = e_pad - e
    if pad:
        src = jnp.concatenate([src, jnp.zeros((pad,), jnp.int32)])
        dst = jnp.concatenate([dst, jnp.full((pad,), n, jnp.int32)])
    x_p = jnp.pad(x, ((0, n_acc - n), (0, 0)))
    b1r = b1.reshape(1, h)
    b2r = b2.reshape(1, c)

    hist = _make_deg_kernel(n_acc, ept)(dst).reshape(NW, n_acc)

    grid = (n_acc // br,)

    g1 = pl.pallas_call(
        _tc1_body,
        grid=grid,
        in_specs=[
            pl.BlockSpec((br, d), lambda i: (i, 0)),
            pl.BlockSpec((d, h), lambda i: (0, 0)),
            pl.BlockSpec((NW, br), lambda i: (0, i)),
        ],
        out_specs=pl.BlockSpec((br, h), lambda i: (i, 0)),
        out_shape=jax.ShapeDtypeStruct((n_acc, h), jnp.float32),
    )(x_p, W1, hist)

    p = _make_agg_kernel(n_acc, ept, nch, h)(g1, src, dst)
    p = p.reshape(NC, n_acc, h)

    g2 = pl.pallas_call(
        _tc2_body,
        grid=grid,
        in_specs=[
            pl.BlockSpec((NC, br, h), lambda i: (0, i, 0)),
            pl.BlockSpec((br, h), lambda i: (i, 0)),
            pl.BlockSpec((NW, br), lambda i: (0, i)),
            pl.BlockSpec((h, c), lambda i: (0, 0)),
            pl.BlockSpec((1, h), lambda i: (0, 0)),
        ],
        out_specs=pl.BlockSpec((br, c), lambda i: (i, 0)),
        out_shape=jax.ShapeDtypeStruct((n_acc, c), jnp.float32),
    )(p, g1, hist, W2, b1r)

    q = _make_agg_kernel(n_acc, ept, nch, c)(g2, src, dst)
    q = q.reshape(NC, n_acc, c)

    out = pl.pallas_call(
        _tc3_body,
        grid=grid,
        in_specs=[
            pl.BlockSpec((NC, br, c), lambda i: (0, i, 0)),
            pl.BlockSpec((br, c), lambda i: (i, 0)),
            pl.BlockSpec((NW, br), lambda i: (0, i)),
            pl.BlockSpec((1, c), lambda i: (0, 0)),
        ],
        out_specs=pl.BlockSpec((br, c), lambda i: (i, 0)),
        out_shape=jax.ShapeDtypeStruct((n_acc, c), jnp.float32),
    )(q, g2, hist, b2r)

    return out[:n]


# speedup vs baseline: 19.8304x; 19.8304x over previous
import functools

import jax
import jax.numpy as jnp
from jax import lax
from jax.experimental import pallas as pl
from jax.experimental.pallas import tpu as pltpu
from jax.experimental.pallas import tpu_sc as plsc

NC = 2
NS = 16
NW = NC * NS
CH = 128


def _round_up(a, b):
    return (a + b - 1) // b * b


def _sc_mesh():
    return plsc.VectorSubcoreMesh(core_axis_name="c", subcore_axis_name="s")


def _make_deg_kernel(n_acc, ept):
    @functools.partial(
        pl.kernel,
        out_type=jax.ShapeDtypeStruct((NW * n_acc,), jnp.float32),
        mesh=_sc_mesh(),
        scratch_types=[
            pltpu.VMEM((n_acc,), jnp.float32),
            pltpu.VMEM((ept,), jnp.int32),
        ],
        compiler_params=pltpu.CompilerParams(needs_layout_passes=False),
    )
    def deg_kernel(dst_hbm, out_hbm, hist_v, dste_v):
        cid = lax.axis_index("c")
        sid = lax.axis_index("s")
        wid = cid * NS + sid
        zero = jnp.zeros((16,), jnp.float32)

        def zbody(i, carry):
            hist_v[pl.ds(i * 16, 16)] = zero
            return carry

        lax.fori_loop(0, n_acc // 16, zbody, 0)

        pltpu.sync_copy(dst_hbm.at[pl.ds(wid * ept, ept)], dste_v)
        ones = jnp.ones((16,), jnp.float32)

        def ebody(i, carry):
            idx = dste_v[pl.ds(i * 16, 16)]
            plsc.addupdate_scatter(hist_v, [idx], ones)
            return carry

        lax.fori_loop(0, ept // 16, ebody, 0)

        pltpu.sync_copy(hist_v, out_hbm.at[pl.ds(wid * n_acc, n_acc)])

    return deg_kernel


def _make_agg_kernel(n_acc, ept, nch, f):
    rpt = n_acc // NS

    @functools.partial(
        pl.kernel,
        out_type=jax.ShapeDtypeStruct((NC * n_acc, f), jnp.float32),
        mesh=_sc_mesh(),
        scratch_types=[
            pltpu.VMEM_SHARED((n_acc, f), jnp.float32),
            pltpu.VMEM((rpt, f), jnp.float32),
            pltpu.VMEM((CH,), jnp.int32),
            pltpu.VMEM((1, CH), jnp.int32),
            pltpu.VMEM((CH, f), jnp.float32),
            pltpu.SemaphoreType.DMA,
        ],
        compiler_params=pltpu.CompilerParams(use_tc_tiling_on_sc=False),
    )
    def agg_kernel(g_hbm, src_hbm, dst_hbm, out_hbm,
                   acc_sh, zrow_v, sidx_v, didx_v, rows_v, sem):
        cid = lax.axis_index("c")
        sid = lax.axis_index("s")
        wid = cid * NS + sid
        zero = jnp.zeros((16,), jnp.float32)

        def zbody(r, carry):
            for j in range(f // 16):
                zrow_v[r, pl.ds(j * 16, 16)] = zero
            return carry

        lax.fori_loop(0, rpt, zbody, 0)
        pltpu.sync_copy(zrow_v, acc_sh.at[pl.ds(sid * rpt, rpt)])
        plsc.subcore_barrier()

        ebase = wid * ept

        def chbody(ci, carry):
            b = ebase + ci * CH
            pltpu.sync_copy(src_hbm.at[pl.ds(b, CH)], sidx_v)
            pltpu.sync_copy(dst_hbm.at[pl.ds(b, CH)], didx_v.at[0])
            pltpu.async_copy(g_hbm.at[sidx_v], rows_v, sem).wait()
            pltpu.sync_copy(rows_v, acc_sh.at[didx_v.at[0]], add=True)
            return carry

        lax.fori_loop(0, nch, chbody, 0)
        plsc.subcore_barrier()

        pltpu.sync_copy(
            acc_sh.at[pl.ds(sid * rpt, rpt)],
            out_hbm.at[pl.ds(cid * n_acc + sid * rpt, rpt)],
        )

    return agg_kernel


def _deg_dinv(hist_blk):
    deg = jnp.sum(hist_blk, axis=0) + 1.0
    return lax.rsqrt(deg)


def _tc1_body(x_ref, w_ref, hist_ref, g1_ref):
    dinv = _deg_dinv(hist_ref[...])
    h = jnp.dot(x_ref[...], w_ref[...], preferred_element_type=jnp.float32)
    g1_ref[...] = h * dinv[:, None]


def _tc2_body(p_ref, g1_ref, hist_ref, w_ref, b_ref, g2_ref):
    dinv = _deg_dinv(hist_ref[...])
    s = (p_ref[0] + p_ref[1] + g1_ref[...]) * dinv[:, None] + b_ref[...]
    a = jnp.maximum(s, 0.0)
    h2 = jnp.dot(a, w_ref[...], preferred_element_type=jnp.float32)
    g2_ref[...] = h2 * dinv[:, None]


def _tc3_body(q_ref, g2_ref, hist_ref, b_ref, out_ref):
    dinv = _deg_dinv(hist_ref[...])
    z = (q_ref[0] + q_ref[1] + g2_ref[...]) * dinv[:, None] + b_ref[...]
    m = jnp.max(z, axis=1, keepdims=True)
    lse = jnp.log(jnp.sum(jnp.exp(z - m), axis=1, keepdims=True))
    out_ref[...] = z - m - lse


def kernel(x, edge_idx, W1, b1, W2, b2):
    n, d = x.shape
    h = W1.shape[1]
    c = W2.shape[1]
    e = edge_idx.shape[1]

    n_acc = _round_up(n + 1, 1024)
    br = 1024
    e_pad = _round_up(e, NW * CH)
    ept = e_pad // NW
    nch = ept // CH

    src = edge_idx[0]
    dst = edge_idx[1]
    pad = e_pad - e
    if pad:
        src = jnp.concatenate([src, jnp.zeros((pad,), jnp.int32)])
        dst = jnp.concatenate([dst, jnp.full((pad,), n, jnp.int32)])
    x_p = jnp.pad(x, ((0, n_acc - n), (0, 0)))
    b1r = b1.reshape(1, h)
    b2r = b2.reshape(1, c)

    hist = _make_deg_kernel(n_acc, ept)(dst).reshape(NW, n_acc)

    grid = (n_acc // br,)

    g1 = pl.pallas_call(
        _tc1_body,
        grid=grid,
        in_specs=[
            pl.BlockSpec((br, d), lambda i: (i, 0)),
            pl.BlockSpec((d, h), lambda i: (0, 0)),
            pl.BlockSpec((NW, br), lambda i: (0, i)),
        ],
        out_specs=pl.BlockSpec((br, h), lambda i: (i, 0)),
        out_shape=jax.ShapeDtypeStruct((n_acc, h), jnp.float32),
    )(x_p, W1, hist)

    p = _make_agg_kernel(n_acc, ept, nch, h)(g1, src, dst)
    p = p.reshape(NC, n_acc, h)

    g2 = pl.pallas_call(
        _tc2_body,
        grid=grid,
        in_specs=[
            pl.BlockSpec((NC, br, h), lambda i: (0, i, 0)),
            pl.BlockSpec((br, h), lambda i: (i, 0)),
            pl.BlockSpec((NW, br), lambda i: (0, i)),
            pl.BlockSpec((h, c), lambda i: (0, 0)),
            pl.BlockSpec((1, h), lambda i: (0, 0)),
        ],
        out_specs=pl.BlockSpec((br, c), lambda i: (i, 0)),
        out_shape=jax.ShapeDtypeStruct((n_acc, c), jnp.float32),
    )(p, g1, hist, W2, b1r)

    q = _make_agg_kernel(n_acc, ept, nch, c)(g2, src, dst)
    q = q.reshape(NC, n_acc, c)

    out = pl.pallas_call(
        _tc3_body,
        grid=grid,
        in_specs=[
            pl.BlockSpec((NC, br, c), lambda i: (0, i, 0)),
            pl.BlockSpec((br, c), lambda i: (i, 0)),
            pl.BlockSpec((NW, br), lambda i: (0, i)),
            pl.BlockSpec((1, c), lambda i: (0, 0)),
        ],
        out_specs=pl.BlockSpec((br, c), lambda i: (i, 0)),
        out_shape=jax.ShapeDtypeStruct((n_acc, c), jnp.float32),
    )(q, g2, hist, b2r)

    return out[:n]
